# baseline (speedup 1.0000x reference)
import jax
import jax.numpy as jnp
from jax import lax
from jax.experimental import pallas as pl
from jax.experimental.pallas import tpu as pltpu
from jax.experimental.pallas import tpu_sc as plsc

B = 16384
V = 100000
F = 64
W = 2 * F
L = 16
NC, NS = 2, 16
NW = NC * NS
BPW = B // NW
CHUNK = 128
N_CHUNKS = BPW // CHUNK
N_BLOCKS = CHUNK // L


def _noop_body(pu_in, qi_in, pu_out, qi_out):
    pass


def _passthrough(pu, qi):
    return pl.pallas_call(
        _noop_body,
        in_specs=[
            pl.BlockSpec(memory_space=pltpu.MemorySpace.HBM),
            pl.BlockSpec(memory_space=pltpu.MemorySpace.HBM),
        ],
        out_specs=[
            pl.BlockSpec(memory_space=pltpu.MemorySpace.HBM),
            pl.BlockSpec(memory_space=pltpu.MemorySpace.HBM),
        ],
        out_shape=[
            jax.ShapeDtypeStruct((V, F), jnp.float32),
            jax.ShapeDtypeStruct((V, F), jnp.float32),
        ],
        input_output_aliases={0: 0, 1: 1},
    )(pu, qi)


def _sc_body(uid_hbm, iid_hbm, pu_hbm, qi_hbm, pred_hbm, feat_hbm,
             uid_v, iid_v, pu_c, qi_c, feat_c, pred_v, sem):
    wid = lax.axis_index("s") * NC + lax.axis_index("c")
    base = wid * BPW

    pltpu.sync_copy(uid_hbm.at[pl.ds(base, BPW)], uid_v)
    pltpu.sync_copy(iid_hbm.at[pl.ds(base, BPW)], iid_v)

    lanes = lax.iota(jnp.int32, L)

    for j in range(N_CHUNKS):
        sl = pl.ds(j * CHUNK, CHUNK)
        cp = pltpu.async_copy(pu_hbm.at[uid_v.at[sl]], pu_c, sem)
        cq = pltpu.async_copy(qi_hbm.at[iid_v.at[sl]], qi_c, sem)
        cp.wait()
        cq.wait()

        def blk_body(blk, _, j=j):
            acc16 = jnp.zeros((L,), jnp.float32)
            for r16 in range(L):
                r = blk * L + r16
                acc = None
                for c in range(F // L):
                    p = pu_c[r, pl.ds(c * L, L)]
                    q = qi_c[r, pl.ds(c * L, L)]
                    feat_c[r, pl.ds(c * L, L)] = p
                    feat_c[r, pl.ds(F + c * L, L)] = q
                    acc = p * q if acc is None else acc + p * q
                s = jnp.sum(acc)
                acc16 = jnp.where(lanes == r16, s, acc16)
            acc16 = jnp.minimum(jnp.maximum(acc16, 1.0), 5.0)
            pred_v[pl.ds(j * CHUNK + blk * L, L)] = acc16
            return 0

        lax.fori_loop(0, N_BLOCKS, blk_body, 0)
        pltpu.sync_copy(feat_c, feat_hbm.at[pl.ds(base + j * CHUNK, CHUNK)])

    pltpu.sync_copy(pred_v, pred_hbm.at[pl.ds(base, BPW)])


def _gather_combine(uid, iid, pu_lin, qi_lin):
    mesh = plsc.VectorSubcoreMesh(core_axis_name="c", subcore_axis_name="s")
    return pl.kernel(
        _sc_body,
        out_type=(
            jax.ShapeDtypeStruct((B,), jnp.float32),
            jax.ShapeDtypeStruct((B, W), jnp.float32),
        ),
        mesh=mesh,
        compiler_params=pltpu.CompilerParams(use_tc_tiling_on_sc=False,
                                             needs_layout_passes=False),
        scratch_types=[
            pltpu.VMEM((BPW,), jnp.int32),
            pltpu.VMEM((BPW,), jnp.int32),
            pltpu.VMEM((CHUNK, F), jnp.float32),
            pltpu.VMEM((CHUNK, F), jnp.float32),
            pltpu.VMEM((CHUNK, W), jnp.float32),
            pltpu.VMEM((BPW,), jnp.float32),
            pltpu.SemaphoreType.DMA,
        ],
    )(uid, iid, pu_lin, qi_lin)


@jax.jit
def _run(user_item, pu, qi):
    pu_lin, qi_lin = _passthrough(pu, qi)
    return _gather_combine(user_item[:, 0], user_item[:, 1], pu_lin, qi_lin)


def kernel(user_item, pu, qi):
    return _run(user_item.astype(jnp.int32), pu, qi)


# speedup vs baseline: 1.1033x; 1.1033x over previous
import jax
import jax.numpy as jnp
from jax import lax
from jax.experimental import pallas as pl
from jax.experimental.pallas import tpu as pltpu
from jax.experimental.pallas import tpu_sc as plsc

B = 16384
V = 100000
F = 64
W = 2 * F
HALF = V // 2
L = 16
NC, NS = 2, 16
NW = NC * NS
BPW = B // NW
CHUNK = 128
N_CHUNKS = BPW // CHUNK
N_BLOCKS = CHUNK // L

PACK_ROWS = 5000
PACK_BLOCKS = HALF // PACK_ROWS


def _pack_body(pu_ref, qi_ref, puw_ref, qiw_ref):
    h = pl.program_id(1)

    @pl.when(h == 0)
    def _():
        puw_ref[:, :F] = pu_ref[...]
        qiw_ref[:, :F] = qi_ref[...]

    @pl.when(h == 1)
    def _():
        puw_ref[:, F:] = pu_ref[...]
        qiw_ref[:, F:] = qi_ref[...]


def _pack(pu, qi):
    in_spec = pl.BlockSpec((PACK_ROWS, F), lambda i, h: (i + h * PACK_BLOCKS, 0))
    out_spec = pl.BlockSpec((PACK_ROWS, W), lambda i, h: (i, 0))
    return pl.pallas_call(
        _pack_body,
        grid=(PACK_BLOCKS, 2),
        in_specs=[in_spec, in_spec],
        out_specs=[out_spec, out_spec],
        out_shape=[
            jax.ShapeDtypeStruct((HALF, W), jnp.float32),
            jax.ShapeDtypeStruct((HALF, W), jnp.float32),
        ],
    )(pu, qi)


def _sc_body(uid_hbm, iid_hbm, puw_hbm, qiw_hbm, pred_hbm, feat_hbm,
             uid_v, iid_v, uwx_v, iwx_v, pu_c, qi_c, feat_c, pred_v, sem):
    wid = lax.axis_index("s") * NC + lax.axis_index("c")
    base = wid * BPW

    pltpu.sync_copy(uid_hbm.at[pl.ds(base, BPW)], uid_v)
    pltpu.sync_copy(iid_hbm.at[pl.ds(base, BPW)], iid_v)

    lanes = lax.iota(jnp.int32, L)

    def wx_body(g, _):
        sl = pl.ds(g * L, L)
        u = uid_v[sl]
        i = iid_v[sl]
        uwx_v[sl] = jnp.where(u >= HALF, u - HALF, u)
        iwx_v[sl] = jnp.where(i >= HALF, i - HALF, i)
        return 0

    lax.fori_loop(0, BPW // L, wx_body, 0)

    for j in range(N_CHUNKS):
        sl = pl.ds(j * CHUNK, CHUNK)
        cp = pltpu.async_copy(puw_hbm.at[uwx_v.at[sl]], pu_c, sem)
        cq = pltpu.async_copy(qiw_hbm.at[iwx_v.at[sl]], qi_c, sem)
        cp.wait()
        cq.wait()

        def blk_body(blk, _, j=j):
            gsl = pl.ds(j * CHUNK + blk * L, L)
            uoffs = jnp.where(uid_v[gsl] >= HALF, F, 0)
            ioffs = jnp.where(iid_v[gsl] >= HALF, F, 0)
            acc16 = jnp.zeros((L,), jnp.float32)
            for r16 in range(L):
                r = blk * L + r16
                uoff = uoffs[r16]
                ioff = ioffs[r16]
                acc = None
                for c in range(F // L):
                    p = pu_c[r, pl.ds(uoff + c * L, L)]
                    q = qi_c[r, pl.ds(ioff + c * L, L)]
                    feat_c[r, pl.ds(c * L, L)] = p
                    feat_c[r, pl.ds(F + c * L, L)] = q
                    acc = p * q if acc is None else acc + p * q
                s = jnp.sum(acc)
                acc16 = jnp.where(lanes == r16, s, acc16)
            acc16 = jnp.minimum(jnp.maximum(acc16, 1.0), 5.0)
            pred_v[pl.ds(j * CHUNK + blk * L, L)] = acc16
            return 0

        lax.fori_loop(0, N_BLOCKS, blk_body, 0)
        pltpu.sync_copy(feat_c, feat_hbm.at[pl.ds(base + j * CHUNK, CHUNK)])

    pltpu.sync_copy(pred_v, pred_hbm.at[pl.ds(base, BPW)])


def _gather_combine(uid, iid, puw, qiw):
    mesh = plsc.VectorSubcoreMesh(core_axis_name="c", subcore_axis_name="s")
    return pl.kernel(
        _sc_body,
        out_type=(
            jax.ShapeDtypeStruct((B,), jnp.float32),
            jax.ShapeDtypeStruct((B, W), jnp.float32),
        ),
        mesh=mesh,
        compiler_params=pltpu.CompilerParams(needs_layout_passes=False),
        scratch_types=[
            pltpu.VMEM((BPW,), jnp.int32),
            pltpu.VMEM((BPW,), jnp.int32),
            pltpu.VMEM((BPW,), jnp.int32),
            pltpu.VMEM((BPW,), jnp.int32),
            pltpu.VMEM((CHUNK, W), jnp.float32),
            pltpu.VMEM((CHUNK, W), jnp.float32),
            pltpu.VMEM((CHUNK, W), jnp.float32),
            pltpu.VMEM((BPW,), jnp.float32),
            pltpu.SemaphoreType.DMA,
        ],
    )(uid, iid, puw, qiw)


@jax.jit
def _run(user_item, pu, qi):
    puw, qiw = _pack(pu, qi)
    return _gather_combine(user_item[:, 0], user_item[:, 1], puw, qiw)


def kernel(user_item, pu, qi):
    return _run(user_item.astype(jnp.int32), pu, qi)


# speedup vs baseline: 1.4828x; 1.3440x over previous
import jax
import jax.numpy as jnp
from jax import lax
from jax.experimental import pallas as pl
from jax.experimental.pallas import tpu as pltpu
from jax.experimental.pallas import tpu_sc as plsc

B = 16384
V = 100000
F = 64
W = 2 * F
HALF = V // 2
L = 16
NC, NS = 2, 16
NW = NC * NS
BPW = B // NW
CHUNK = 128
N_CHUNKS = BPW // CHUNK
N_BLOCKS = CHUNK // L

PACK_ROWS = 5000
PACK_BLOCKS = HALF // PACK_ROWS


def _pack_body(t_ref, out_ref):
    out_ref[...] = jnp.concatenate([t_ref[0], t_ref[1]], axis=1)


def _pack_one(table):
    return pl.pallas_call(
        _pack_body,
        grid=(PACK_BLOCKS,),
        in_specs=[pl.BlockSpec((2, PACK_ROWS, F), lambda i: (0, i, 0))],
        out_specs=pl.BlockSpec((PACK_ROWS, W), lambda i: (i, 0)),
        out_shape=jax.ShapeDtypeStruct((HALF, W), jnp.float32),
    )(table.reshape(2, HALF, F))


def _sc_body(uid_hbm, iid_hbm, puw_hbm, qiw_hbm, pred_hbm, feat_hbm,
             uid_v, iid_v, uwx_v, iwx_v, pu_c, qi_c, feat_c, pred_v, sem):
    wid = lax.axis_index("s") * NC + lax.axis_index("c")
    base = wid * BPW

    pltpu.sync_copy(uid_hbm.at[pl.ds(base, BPW)], uid_v)
    pltpu.sync_copy(iid_hbm.at[pl.ds(base, BPW)], iid_v)

    lanes = lax.iota(jnp.int32, L)

    def wx_body(g, _):
        sl = pl.ds(g * L, L)
        u = uid_v[sl]
        i = iid_v[sl]
        uwx_v[sl] = jnp.where(u >= HALF, u - HALF, u)
        iwx_v[sl] = jnp.where(i >= HALF, i - HALF, i)
        return 0

    lax.fori_loop(0, BPW // L, wx_body, 0)

    def gather_chunk(j, buf):
        sl = pl.ds(j * CHUNK, CHUNK)
        cp = pltpu.async_copy(puw_hbm.at[uwx_v.at[sl]], pu_c.at[buf], sem)
        cq = pltpu.async_copy(qiw_hbm.at[iwx_v.at[sl]], qi_c.at[buf], sem)
        return cp, cq

    def compute_chunk(j, buf):
        def blk_body(blk, _):
            gsl = pl.ds(j * CHUNK + blk * L, L)
            uoffs = jnp.where(uid_v[gsl] >= HALF, F, 0)
            ioffs = jnp.where(iid_v[gsl] >= HALF, F, 0)
            acc16 = jnp.zeros((L,), jnp.float32)
            for r16 in range(L):
                r = blk * L + r16
                uoff = uoffs[r16]
                ioff = ioffs[r16]
                acc = None
                for c in range(F // L):
                    p = pu_c[buf, r, pl.ds(uoff + c * L, L)]
                    q = qi_c[buf, r, pl.ds(ioff + c * L, L)]
                    feat_c[r, pl.ds(c * L, L)] = p
                    feat_c[r, pl.ds(F + c * L, L)] = q
                    acc = p * q if acc is None else acc + p * q
                s = jnp.sum(acc)
                acc16 = jnp.where(lanes == r16, s, acc16)
            acc16 = jnp.minimum(jnp.maximum(acc16, 1.0), 5.0)
            pred_v[pl.ds(j * CHUNK + blk * L, L)] = acc16
            return 0

        lax.fori_loop(0, N_BLOCKS, blk_body, 0)
        pltpu.sync_copy(feat_c, feat_hbm.at[pl.ds(base + j * CHUNK, CHUNK)])

    pending = gather_chunk(0, 0)
    for j in range(N_CHUNKS):
        buf = j % 2
        pending[0].wait()
        pending[1].wait()
        if j + 1 < N_CHUNKS:
            nxt = gather_chunk(j + 1, (j + 1) % 2)
        compute_chunk(j, buf)
        if j + 1 < N_CHUNKS:
            pending = nxt

    pltpu.sync_copy(pred_v, pred_hbm.at[pl.ds(base, BPW)])


def _gather_combine(uid, iid, puw, qiw):
    mesh = plsc.VectorSubcoreMesh(core_axis_name="c", subcore_axis_name="s")
    return pl.kernel(
        _sc_body,
        out_type=(
            jax.ShapeDtypeStruct((B,), jnp.float32),
            jax.ShapeDtypeStruct((B, W), jnp.float32),
        ),
        mesh=mesh,
        compiler_params=pltpu.CompilerParams(needs_layout_passes=False),
        scratch_types=[
            pltpu.VMEM((BPW,), jnp.int32),
            pltpu.VMEM((BPW,), jnp.int32),
            pltpu.VMEM((BPW,), jnp.int32),
            pltpu.VMEM((BPW,), jnp.int32),
            pltpu.VMEM((2, CHUNK, W), jnp.float32),
            pltpu.VMEM((2, CHUNK, W), jnp.float32),
            pltpu.VMEM((CHUNK, W), jnp.float32),
            pltpu.VMEM((BPW,), jnp.float32),
            pltpu.SemaphoreType.DMA,
        ],
    )(uid, iid, puw, qiw)


@jax.jit
def _run(user_item, pu, qi):
    puw = _pack_one(pu)
    qiw = _pack_one(qi)
    return _gather_combine(user_item[:, 0], user_item[:, 1], puw, qiw)


def kernel(user_item, pu, qi):
    return _run(user_item.astype(jnp.int32), pu, qi)


# speedup vs baseline: 1.4950x; 1.0082x over previous
import jax
import jax.numpy as jnp
from jax import lax
from jax.experimental import pallas as pl
from jax.experimental.pallas import tpu as pltpu
from jax.experimental.pallas import tpu_sc as plsc

B = 16384
V = 100000
F = 64
W = 2 * F
HALF = V // 2
L = 16
NC, NS = 2, 16
NW = NC * NS
BPW = B // NW
CHUNK = 128
N_CHUNKS = BPW // CHUNK
N_BLOCKS = CHUNK // L

PACK_ROWS = 10000
PACK_BLOCKS = HALF // PACK_ROWS


def _pack_body(t_ref, out_ref):
    out_ref[...] = jnp.concatenate([t_ref[0], t_ref[1]], axis=1)


def _pack_one(table):
    return pl.pallas_call(
        _pack_body,
        grid=(PACK_BLOCKS,),
        in_specs=[pl.BlockSpec((2, PACK_ROWS, F), lambda i: (0, i, 0))],
        out_specs=pl.BlockSpec((PACK_ROWS, W), lambda i: (i, 0)),
        out_shape=jax.ShapeDtypeStruct((HALF, W), jnp.float32),
    )(table.reshape(2, HALF, F))


def _sc_body(uid_hbm, iid_hbm, puw_hbm, qiw_hbm, pred_hbm, feat_hbm,
             uid_v, iid_v, uwx_v, iwx_v, pu_c, qi_c, feat_c, pred_v, sem,
             fsem):
    wid = lax.axis_index("s") * NC + lax.axis_index("c")
    base = wid * BPW

    pltpu.sync_copy(uid_hbm.at[pl.ds(base, BPW)], uid_v)
    pltpu.sync_copy(iid_hbm.at[pl.ds(base, BPW)], iid_v)

    lanes = lax.iota(jnp.int32, L)

    def wx_body(g, _):
        sl = pl.ds(g * L, L)
        u = uid_v[sl]
        i = iid_v[sl]
        uwx_v[sl] = jnp.where(u >= HALF, u - HALF, u)
        iwx_v[sl] = jnp.where(i >= HALF, i - HALF, i)
        return 0

    lax.fori_loop(0, BPW // L, wx_body, 0)

    def gather_chunk(j, buf):
        sl = pl.ds(j * CHUNK, CHUNK)
        cp = pltpu.async_copy(puw_hbm.at[uwx_v.at[sl]], pu_c.at[buf], sem)
        cq = pltpu.async_copy(qiw_hbm.at[iwx_v.at[sl]], qi_c.at[buf], sem)
        return cp, cq

    def compute_chunk(j, buf):
        fb = feat_c.at[buf]

        def blk_body(blk, _):
            gsl = pl.ds(j * CHUNK + blk * L, L)
            uoffs = jnp.where(uid_v[gsl] >= HALF, F, 0)
            ioffs = jnp.where(iid_v[gsl] >= HALF, F, 0)
            acc16 = jnp.zeros((L,), jnp.float32)
            for r16 in range(L):
                r = blk * L + r16
                uoff = uoffs[r16]
                ioff = ioffs[r16]
                acc = None
                for c in range(F // L):
                    p = pu_c[buf, r, pl.ds(uoff + c * L, L)]
                    q = qi_c[buf, r, pl.ds(ioff + c * L, L)]
                    feat_c[buf, r, pl.ds(c * L, L)] = p
                    feat_c[buf, r, pl.ds(F + c * L, L)] = q
                    acc = p * q if acc is None else acc + p * q
                s = jnp.sum(acc)
                acc16 = jnp.where(lanes == r16, s, acc16)
            acc16 = jnp.minimum(jnp.maximum(acc16, 1.0), 5.0)
            pred_v[pl.ds(j * CHUNK + blk * L, L)] = acc16
            return 0

        lax.fori_loop(0, N_BLOCKS, blk_body, 0)
        return pltpu.async_copy(
            fb, feat_hbm.at[pl.ds(base + j * CHUNK, CHUNK)], fsem)

    pending = gather_chunk(0, 0)
    fcopies = []
    for j in range(N_CHUNKS):
        buf = j % 2
        pending[0].wait()
        pending[1].wait()
        if j + 1 < N_CHUNKS:
            nxt = gather_chunk(j + 1, (j + 1) % 2)
        if j >= 2:
            fcopies[j - 2].wait()
        fcopies.append(compute_chunk(j, buf))
        if j + 1 < N_CHUNKS:
            pending = nxt

    pltpu.sync_copy(pred_v, pred_hbm.at[pl.ds(base, BPW)])
    for fc in fcopies[-2:]:
        fc.wait()


def _gather_combine(uid, iid, puw, qiw):
    mesh = plsc.VectorSubcoreMesh(core_axis_name="c", subcore_axis_name="s")
    return pl.kernel(
        _sc_body,
        out_type=(
            jax.ShapeDtypeStruct((B,), jnp.float32),
            jax.ShapeDtypeStruct((B, W), jnp.float32),
        ),
        mesh=mesh,
        compiler_params=pltpu.CompilerParams(needs_layout_passes=False),
        scratch_types=[
            pltpu.VMEM((BPW,), jnp.int32),
            pltpu.VMEM((BPW,), jnp.int32),
            pltpu.VMEM((BPW,), jnp.int32),
            pltpu.VMEM((BPW,), jnp.int32),
            pltpu.VMEM((2, CHUNK, W), jnp.float32),
            pltpu.VMEM((2, CHUNK, W), jnp.float32),
            pltpu.VMEM((2, CHUNK, W), jnp.float32),
            pltpu.VMEM((BPW,), jnp.float32),
            pltpu.SemaphoreType.DMA,
            pltpu.SemaphoreType.DMA,
        ],
    )(uid, iid, puw, qiw)


@jax.jit
def _run(user_item, pu, qi):
    puw = _pack_one(pu)
    qiw = _pack_one(qi)
    return _gather_combine(user_item[:, 0], user_item[:, 1], puw, qiw)


def kernel(user_item, pu, qi):
    return _run(user_item.astype(jnp.int32), pu, qi)


# speedup vs baseline: 1.4977x; 1.0018x over previous
import jax
import jax.numpy as jnp
from jax import lax
from jax.experimental import pallas as pl
from jax.experimental.pallas import tpu as pltpu
from jax.experimental.pallas import tpu_sc as plsc

B = 16384
V = 100000
F = 64
W = 2 * F
HALF = V // 2
L = 16
NC, NS = 2, 16
NW = NC * NS
BPW = B // NW
CHUNK = 128
N_CHUNKS = BPW // CHUNK
N_BLOCKS = CHUNK // L

PACK_ROWS = 5000
PACK_BLOCKS = HALF // PACK_ROWS


def _pack_body(t_ref, out_ref):
    out_ref[...] = jnp.concatenate([t_ref[0], t_ref[1]], axis=1)


def _pack_one(table):
    return pl.pallas_call(
        _pack_body,
        grid=(PACK_BLOCKS,),
        in_specs=[pl.BlockSpec((2, PACK_ROWS, F), lambda i: (0, i, 0))],
        out_specs=pl.BlockSpec((PACK_ROWS, W), lambda i: (i, 0)),
        out_shape=jax.ShapeDtypeStruct((HALF, W), jnp.float32),
    )(table.reshape(2, HALF, F))


def _sc_body(uid_hbm, iid_hbm, puw_hbm, qiw_hbm, pred_hbm, feat_hbm,
             uid_v, iid_v, uwx_v, iwx_v, pu_c, qi_c, feat_c, pred_v, sem,
             fsem):
    wid = lax.axis_index("s") * NC + lax.axis_index("c")
    base = wid * BPW

    pltpu.sync_copy(uid_hbm.at[pl.ds(base, BPW)], uid_v)
    pltpu.sync_copy(iid_hbm.at[pl.ds(base, BPW)], iid_v)

    lanes = lax.iota(jnp.int32, L)

    def wx_body(g, _):
        sl = pl.ds(g * L, L)
        u = uid_v[sl]
        i = iid_v[sl]
        uwx_v[sl] = jnp.where(u >= HALF, u - HALF, u)
        iwx_v[sl] = jnp.where(i >= HALF, i - HALF, i)
        return 0

    lax.fori_loop(0, BPW // L, wx_body, 0)

    def gather_chunk(j, buf):
        sl = pl.ds(j * CHUNK, CHUNK)
        cp = pltpu.async_copy(puw_hbm.at[uwx_v.at[sl]], pu_c.at[buf], sem)
        cq = pltpu.async_copy(qiw_hbm.at[iwx_v.at[sl]], qi_c.at[buf], sem)
        return cp, cq

    def compute_chunk(j, buf):
        fb = feat_c.at[buf]

        def blk_body(blk, _):
            gsl = pl.ds(j * CHUNK + blk * L, L)
            uoffs = jnp.where(uid_v[gsl] >= HALF, F, 0)
            ioffs = jnp.where(iid_v[gsl] >= HALF, F, 0)
            acc16 = jnp.zeros((L,), jnp.float32)
            for r16 in range(L):
                r = blk * L + r16
                uoff = uoffs[r16]
                ioff = ioffs[r16]
                acc = None
                for c in range(F // L):
                    p = pu_c[buf, r, pl.ds(uoff + c * L, L)]
                    q = qi_c[buf, r, pl.ds(ioff + c * L, L)]
                    feat_c[buf, r, pl.ds(c * L, L)] = p
                    feat_c[buf, r, pl.ds(F + c * L, L)] = q
                    acc = p * q if acc is None else acc + p * q
                s = jnp.sum(acc)
                acc16 = jnp.where(lanes == r16, s, acc16)
            acc16 = jnp.minimum(jnp.maximum(acc16, 1.0), 5.0)
            pred_v[pl.ds(j * CHUNK + blk * L, L)] = acc16
            return 0

        lax.fori_loop(0, N_BLOCKS, blk_body, 0)
        return pltpu.async_copy(
            fb, feat_hbm.at[pl.ds(base + j * CHUNK, CHUNK)], fsem)

    pending = gather_chunk(0, 0)
    fcopies = []
    for j in range(N_CHUNKS):
        buf = j % 2
        pending[0].wait()
        pending[1].wait()
        if j + 1 < N_CHUNKS:
            nxt = gather_chunk(j + 1, (j + 1) % 2)
        if j >= 2:
            fcopies[j - 2].wait()
        fcopies.append(compute_chunk(j, buf))
        if j + 1 < N_CHUNKS:
            pending = nxt

    pltpu.sync_copy(pred_v, pred_hbm.at[pl.ds(base, BPW)])
    for fc in fcopies[-2:]:
        fc.wait()


def _gather_combine(uid, iid, puw, qiw):
    mesh = plsc.VectorSubcoreMesh(core_axis_name="c", subcore_axis_name="s")
    return pl.kernel(
        _sc_body,
        out_type=(
            jax.ShapeDtypeStruct((B,), jnp.float32),
            jax.ShapeDtypeStruct((B, W), jnp.float32),
        ),
        mesh=mesh,
        compiler_params=pltpu.CompilerParams(needs_layout_passes=False),
        scratch_types=[
            pltpu.VMEM((BPW,), jnp.int32),
            pltpu.VMEM((BPW,), jnp.int32),
            pltpu.VMEM((BPW,), jnp.int32),
            pltpu.VMEM((BPW,), jnp.int32),
            pltpu.VMEM((2, CHUNK, W), jnp.float32),
            pltpu.VMEM((2, CHUNK, W), jnp.float32),
            pltpu.VMEM((2, CHUNK, W), jnp.float32),
            pltpu.VMEM((BPW,), jnp.float32),
            pltpu.SemaphoreType.DMA,
            pltpu.SemaphoreType.DMA,
        ],
    )(uid, iid, puw, qiw)


@jax.jit
def _run(user_item, pu, qi):
    puw = _pack_one(pu)
    qiw = _pack_one(qi)
    return _gather_combine(user_item[:, 0], user_item[:, 1], puw, qiw)


def kernel(user_item, pu, qi):
    return _run(user_item.astype(jnp.int32), pu, qi)
